# baseline (speedup 1.0000x reference)
import functools

import jax
import jax.numpy as jnp
from jax import lax
from jax.experimental import pallas as pl
from jax.experimental.pallas import tpu as pltpu
from jax.experimental.pallas import tpu_sc as plsc

H = 128
MID = 64


def _swish(v):
    return v * jax.nn.sigmoid(v)


def _pre_body(x_ref, wh1_ref, bh1_ref, wh2_ref, bh2_ref, x1_ref, x2_ref):
    xb = x_ref[...]
    x1_ref[...] = _swish(
        jnp.dot(xb, wh1_ref[...], preferred_element_type=jnp.float32)
        + bh1_ref[...])
    x2_ref[...] = _swish(
        jnp.dot(xb, wh2_ref[...], preferred_element_type=jnp.float32)
        + bh2_ref[...])


def _pre(x, wh1, bh1, wh2, bh2):
    n = x.shape[0]
    bn = 1000
    g = n // bn
    full = lambda i: (0, 0)
    nspec = pl.BlockSpec((bn, H), lambda i: (i, 0))
    return pl.pallas_call(
        _pre_body,
        grid=(g,),
        in_specs=[
            nspec,
            pl.BlockSpec((H, H), full),
            pl.BlockSpec((1, H), full),
            pl.BlockSpec((H, H), full),
            pl.BlockSpec((1, H), full),
        ],
        out_specs=[nspec, nspec],
        out_shape=[
            jax.ShapeDtypeStruct((n, H), jnp.float32),
            jax.ShapeDtypeStruct((n, H), jnp.float32),
        ],
    )(x, wh1, bh1, wh2, bh2)


def _ew_body(f0_ref, f1_ref, f2_ref,
             w01_ref, w02_ref, w11_ref, w12_ref, w21_ref, w22_ref,
             e0_ref, e1_ref, e2_ref):
    for fr, wa, wb, er in (
        (f0_ref, w01_ref, w02_ref, e0_ref),
        (f1_ref, w11_ref, w12_ref, e1_ref),
        (f2_ref, w21_ref, w22_ref, e2_ref),
    ):
        mid = jnp.dot(fr[...], wa[...], preferred_element_type=jnp.float32)
        er[...] = jnp.dot(mid, wb[...], preferred_element_type=jnp.float32)


def _ew_one(feat, w1, w2):
    e = feat.shape[0]
    be = 2000
    g = e // be
    f = feat.shape[1]
    full = lambda i: (0, 0)
    return pl.pallas_call(
        lambda fr, wa, wb, er: _ew_body_one(fr, wa, wb, er),
        grid=(g,),
        in_specs=[
            pl.BlockSpec((be, f), lambda i: (i, 0)),
            pl.BlockSpec((f, MID), full),
            pl.BlockSpec((MID, H), full),
        ],
        out_specs=pl.BlockSpec((be, H), lambda i: (i, 0)),
        out_shape=jax.ShapeDtypeStruct((e, H), jnp.float32),
    )(feat, w1, w2)


def _ew_body_one(fr, wa, wb, er):
    mid = jnp.dot(fr[...], wa[...], preferred_element_type=jnp.float32)
    er[...] = jnp.dot(mid, wb[...], preferred_element_type=jnp.float32)


def _ew_two(fa, wa1, wa2, fb, wb1, wb2):
    e = fa.shape[0]
    be = 2000
    g = e // be
    dfa, dfb = fa.shape[1], fb.shape[1]
    full = lambda i: (0, 0)
    espec = pl.BlockSpec((be, H), lambda i: (i, 0))
    eshape = jax.ShapeDtypeStruct((e, H), jnp.float32)

    def body(fra, waa, wab, frb, wba, wbb, era, erb):
        _ew_body_one(fra, waa, wab, era)
        _ew_body_one(frb, wba, wbb, erb)

    return pl.pallas_call(
        body,
        grid=(g,),
        in_specs=[
            pl.BlockSpec((be, dfa), lambda i: (i, 0)),
            pl.BlockSpec((dfa, MID), full),
            pl.BlockSpec((MID, H), full),
            pl.BlockSpec((be, dfb), lambda i: (i, 0)),
            pl.BlockSpec((dfb, MID), full),
            pl.BlockSpec((MID, H), full),
        ],
        out_specs=[espec, espec],
        out_shape=[eshape, eshape],
    )(fa, wa1, wa2, fb, wb1, wb2)


def _ew(feature0, feature1, pos_emb, p):
    e = feature0.shape[0]
    be = 2000
    g = e // be
    f0, f1, f2 = feature0.shape[1], feature1.shape[1], pos_emb.shape[1]
    full = lambda i: (0, 0)
    espec = pl.BlockSpec((be, H), lambda i: (i, 0))
    eshape = jax.ShapeDtypeStruct((e, H), jnp.float32)
    return pl.pallas_call(
        _ew_body,
        grid=(g,),
        in_specs=[
            pl.BlockSpec((be, f0), lambda i: (i, 0)),
            pl.BlockSpec((be, f1), lambda i: (i, 0)),
            pl.BlockSpec((be, f2), lambda i: (i, 0)),
            pl.BlockSpec((f0, MID), full),
            pl.BlockSpec((MID, H), full),
            pl.BlockSpec((f1, MID), full),
            pl.BlockSpec((MID, H), full),
            pl.BlockSpec((f2, MID), full),
            pl.BlockSpec((MID, H), full),
        ],
        out_specs=[espec, espec, espec],
        out_shape=[eshape, eshape, eshape],
    )(feature0, feature1, pos_emb,
      p["f0_w1"], p["f0_w2"], p["f1_w1"], p["f1_w2"], p["f2_w1"], p["f2_w2"])


@functools.lru_cache(maxsize=None)
def _make_sc(n, e, nk=3):
    nc = 2
    nt = 16
    epc = e // nc
    ept = epc // nt
    ch = 48
    nfull = ept // ch
    tl = ept - nfull * ch
    ntrip = (nfull - 1) // 3
    assert nfull - 1 == 3 * ntrip and ntrip >= 2 and tl % 8 == 0
    n_pad = ((n + nt * 8 - 1) // (nt * 8)) * nt * 8
    npt = n_pad // nt
    mesh = plsc.VectorSubcoreMesh(core_axis_name="c", subcore_axis_name="s")
    out_t = jax.ShapeDtypeStruct((nc, n_pad, H), jnp.float32)

    @functools.partial(
        pl.kernel,
        mesh=mesh,
        out_type=[out_t] * nk,
        scratch_types=[
            pltpu.VMEM((ept,), jnp.int32),
            pltpu.VMEM((ch,), jnp.int32),
            pltpu.VMEM((ch,), jnp.int32),
            pltpu.VMEM((ch,), jnp.int32),
            pltpu.VMEM((tl,), jnp.int32),
            pltpu.VMEM((ch, H), jnp.float32),
            pltpu.VMEM((ch, H), jnp.float32),
            pltpu.VMEM((ch, H), jnp.float32),
            pltpu.VMEM((ch, H), jnp.float32),
            pltpu.VMEM((ch, H), jnp.float32),
            pltpu.VMEM((ch, H), jnp.float32),
            pltpu.VMEM_SHARED((n_pad, H), jnp.float32),
        ] + [pltpu.SemaphoreType.DMA] * 12,
    )
    def sc_conv(*refs):
        src_hbm, dst_hbm, zeros_hbm, x1_hbm = refs[:4]
        ew_hbms = refs[4:4 + nk]
        outs = refs[4 + nk:4 + 2 * nk]
        rest = refs[4 + 2 * nk:]
        srcall = rest[0]
        dstb = rest[1:4]
        dstT = rest[4]
        xr = rest[5:8]
        ewb = rest[8:11]
        acc = rest[11]
        gs = rest[12:15]
        es = rest[15:18]
        ds_ = rest[18:21]
        ss = rest[21:24]
        c = lax.axis_index("c")
        s = lax.axis_index("s")
        rows = pl.ds(pl.multiple_of(s * npt, 8), npt)
        tbase = pl.multiple_of(c * epc + s * ept, 8)

        pltpu.sync_copy(src_hbm.at[pl.ds(tbase, ept)], srcall)

        def issue_loads(ew_hbm, ci, q):
            off = pl.multiple_of(ci * ch, 8)
            pltpu.make_async_copy(
                x1_hbm.at[srcall.at[pl.ds(off, ch)]], xr[q], gs[q]).start()
            ebase = pl.multiple_of(tbase + ci * ch, 8)
            pltpu.make_async_copy(
                ew_hbm.at[pl.ds(ebase, ch)], ewb[q], es[q]).start()
            pltpu.make_async_copy(
                dst_hbm.at[pl.ds(ebase, ch)], dstb[q], ds_[q]).start()

        def multiply(xrb, eb, m):
            @plsc.parallel_loop(0, m, 1, unroll=2)
            def _(r):
                for j in range(H // 16):
                    sl = pl.ds(j * 16, 16)
                    eb[r, sl] = eb[r, sl] * xrb[r, sl]

        def scatter_start(q):
            pltpu.async_copy(ewb[q], acc.at[dstb[q]], ss[q], add=True)

        def scatter_wait(q):
            pltpu.make_async_copy(ewb[q], acc.at[dstb[q]], ss[q]).wait()

        def process(ew_hbm, ci, q, qn, issue_ci, do_wait, do_issue):
            pltpu.make_async_copy(
                x1_hbm.at[srcall.at[pl.ds(0, ch)]], xr[q], gs[q]).wait()
            pltpu.make_async_copy(
                ew_hbm.at[pl.ds(0, ch)], ewb[q], es[q]).wait()
            multiply(xr[q], ewb[q], ch)
            pltpu.make_async_copy(
                dst_hbm.at[pl.ds(0, ch)], dstb[q], ds_[q]).wait()
            scatter_start(q)
            if do_wait:
                scatter_wait(qn)
            if do_issue:
                issue_loads(ew_hbm, issue_ci, qn)

        for ew_hbm, out in zip(ew_hbms, outs):
            issue_loads(ew_hbm, 0, 0)
            issue_loads(ew_hbm, 1, 1)
            pltpu.sync_copy(zeros_hbm, acc.at[rows])
            plsc.subcore_barrier()

            process(ew_hbm, 0, 0, 2, 2, False, True)

            def triple(i, carry, ew_hbm=ew_hbm):
                j = 3 * i + 1
                process(ew_hbm, j, 1, 0, j + 2, True, True)
                process(ew_hbm, j + 1, 2, 1, j + 3, True, True)
                process(ew_hbm, j + 2, 0, 2, j + 4, True, True)
                return carry

            lax.fori_loop(0, ntrip - 1, triple, 0)
            jl = nfull - 3
            process(ew_hbm, jl, 1, 0, jl + 2, True, True)
            process(ew_hbm, jl + 1, 2, 1, 0, True, False)
            process(ew_hbm, jl + 2, 0, 2, 0, True, False)
            scatter_wait(0)

            toff = nfull * ch
            tb = pl.multiple_of(tbase + toff, 8)
            pltpu.sync_copy(dst_hbm.at[pl.ds(tb, tl)], dstT)
            pltpu.async_copy(
                x1_hbm.at[srcall.at[pl.ds(toff, tl)]],
                xr[0].at[pl.ds(0, tl)], gs[0]).wait()
            pltpu.sync_copy(ew_hbm.at[pl.ds(tb, tl)], ewb[0].at[pl.ds(0, tl)])
            multiply(xr[0], ewb[0], tl)
            pltpu.sync_copy(ewb[0].at[pl.ds(0, tl)], acc.at[dstT], add=True)

            plsc.subcore_barrier()
            pltpu.sync_copy(acc.at[rows], out.at[c, rows])

    return sc_conv


def _tail_body(a0_ref, a1_ref, a2_ref, x1_ref, x2_ref,
               wrel0, brel0, wroot0, wo0, bo0,
               wrel1, brel1, wroot1, wo1, bo1,
               wrel2, brel2, wroot2, wo2, bo2,
               wc0, bc0, wc1, bc1, wc2, bc2,
               ws0, bs0, ws1, bs1, wfin, bfin,
               out_ref):
    x1 = x1_ref[...]
    x2 = x2_ref[...]
    hs = []
    for ar, wrel, brel, wroot, wo, bo in (
        (a0_ref, wrel0, brel0, wroot0, wo0, bo0),
        (a1_ref, wrel1, brel1, wroot1, wo1, bo1),
        (a2_ref, wrel2, brel2, wroot2, wo2, bo2),
    ):
        agg = ar[0] + ar[1]
        t = (jnp.dot(agg, wrel[...], preferred_element_type=jnp.float32)
             + jnp.dot(x1, wroot[...], preferred_element_type=jnp.float32)
             + brel[...])
        hs.append(_swish(jnp.dot(t, wo[...], preferred_element_type=jnp.float32)
                         + bo[...]))
    wc0v = wc0[...]
    u = (jnp.dot(hs[0], wc0v[:H, :], preferred_element_type=jnp.float32)
         + jnp.dot(hs[1], wc0v[H:2 * H, :], preferred_element_type=jnp.float32)
         + jnp.dot(hs[2], wc0v[2 * H:, :], preferred_element_type=jnp.float32)
         + bc0[...])
    h = _swish(u)
    h = _swish(jnp.dot(h, wc1[...], preferred_element_type=jnp.float32) + bc1[...])
    h = _swish(jnp.dot(h, wc2[...], preferred_element_type=jnp.float32) + bc2[...])
    h = h + x2
    h = _swish(jnp.dot(h, ws0[...], preferred_element_type=jnp.float32) + bs0[...])
    h = _swish(jnp.dot(h, ws1[...], preferred_element_type=jnp.float32) + bs1[...])
    out_ref[...] = jnp.dot(h, wfin[...], preferred_element_type=jnp.float32) + bfin[...]


def _tail(a0, a1, a2, x1, x2, p):
    n = x1.shape[0]
    bn = 1000
    g = n // bn
    full = lambda i: (0, 0)
    aspec = pl.BlockSpec((2, bn, H), lambda i: (0, i, 0))
    nspec = pl.BlockSpec((bn, H), lambda i: (i, 0))
    wspec = pl.BlockSpec((H, H), full)
    bspec = pl.BlockSpec((1, H), full)
    b2 = lambda v: v.reshape(1, H)
    args = [a0, a1, a2, x1, x2]
    specs = [aspec, aspec, aspec, nspec, nspec]
    for k in ("c0", "c1", "c2"):
        args += [p[k + "_wrel"], b2(p[k + "_brel"]), p[k + "_wroot"]]
        specs += [wspec, bspec, wspec]
        o = "o" + k[1]
        args += [p["w_" + o], b2(p["b_" + o])]
        specs += [wspec, bspec]
    args += [p["wc0"], b2(p["bc0"]), p["wc1"], b2(p["bc1"]),
             p["wc2"], b2(p["bc2"]), p["ws0"], b2(p["bs0"]),
             p["ws1"], b2(p["bs1"]), p["w_fin"], b2(p["b_fin"])]
    specs += [pl.BlockSpec((3 * H, H), full), bspec, wspec, bspec,
              wspec, bspec, wspec, bspec, wspec, bspec, wspec, bspec]
    return pl.pallas_call(
        _tail_body,
        grid=(g,),
        in_specs=specs,
        out_specs=pl.BlockSpec((bn, H), lambda i: (i, 0)),
        out_shape=jax.ShapeDtypeStruct((n, H), jnp.float32),
    )(*args)


def kernel(x, feature0, feature1, pos_emb, edge_index, batch, params):
    del batch
    p = params
    n, e = x.shape[0], feature0.shape[0]
    x1, x2 = _pre(x, p["w_h1"], p["b_h1"].reshape(1, H),
                  p["w_h2"], p["b_h2"].reshape(1, H))
    n_pad = ((n + 127) // 128) * 128
    zeros = jnp.zeros((n_pad // 16, H), jnp.float32)
    sc1 = _make_sc(n, e, 1)
    sc2 = _make_sc(n, e, 2)
    src, dst = edge_index[0], edge_index[1]
    ew0 = _ew_one(feature0, p["f0_w1"], p["f0_w2"])
    (a0,) = sc1(src, dst, zeros, x1, ew0)
    ew1, ew2 = _ew_two(feature1, p["f1_w1"], p["f1_w2"],
                       pos_emb, p["f2_w1"], p["f2_w2"])
    a1, a2 = sc2(src, dst, zeros, x1, ew1, ew2)
    return _tail(a0, a1, a2, x1, x2, p)


# speedup vs baseline: 1.0094x; 1.0094x over previous
import functools

import jax
import jax.numpy as jnp
from jax import lax
from jax.experimental import pallas as pl
from jax.experimental.pallas import tpu as pltpu
from jax.experimental.pallas import tpu_sc as plsc

H = 128
MID = 64


def _swish(v):
    return v * jax.nn.sigmoid(v)


def _pre_body(x_ref, wh1_ref, bh1_ref, wh2_ref, bh2_ref, x1_ref, x2_ref):
    xb = x_ref[...]
    x1_ref[...] = _swish(
        jnp.dot(xb, wh1_ref[...], preferred_element_type=jnp.float32)
        + bh1_ref[...])
    x2_ref[...] = _swish(
        jnp.dot(xb, wh2_ref[...], preferred_element_type=jnp.float32)
        + bh2_ref[...])


def _pre(x, wh1, bh1, wh2, bh2):
    n = x.shape[0]
    bn = 1000
    g = n // bn
    full = lambda i: (0, 0)
    nspec = pl.BlockSpec((bn, H), lambda i: (i, 0))
    return pl.pallas_call(
        _pre_body,
        grid=(g,),
        in_specs=[
            nspec,
            pl.BlockSpec((H, H), full),
            pl.BlockSpec((1, H), full),
            pl.BlockSpec((H, H), full),
            pl.BlockSpec((1, H), full),
        ],
        out_specs=[nspec, nspec],
        out_shape=[
            jax.ShapeDtypeStruct((n, H), jnp.float32),
            jax.ShapeDtypeStruct((n, H), jnp.float32),
        ],
    )(x, wh1, bh1, wh2, bh2)


def _ew_one(feat, w1, w2):
    e = feat.shape[0]
    be = 2000
    g = e // be
    f = feat.shape[1]
    full = lambda i: (0, 0)
    return pl.pallas_call(
        lambda fr, wa, wb, er: _ew_body_one(fr, wa, wb, er),
        grid=(g,),
        in_specs=[
            pl.BlockSpec((be, f), lambda i: (i, 0)),
            pl.BlockSpec((f, MID), full),
            pl.BlockSpec((MID, H), full),
        ],
        out_specs=pl.BlockSpec((be, H), lambda i: (i, 0)),
        out_shape=jax.ShapeDtypeStruct((e, H), jnp.float32),
    )(feat, w1, w2)


def _ew_body_one(fr, wa, wb, er):
    mid = jnp.dot(fr[...], wa[...], preferred_element_type=jnp.float32)
    er[...] = jnp.dot(mid, wb[...], preferred_element_type=jnp.float32)


def _ew_two(fa, wa1, wa2, fb, wb1, wb2):
    e = fa.shape[0]
    be = 2000
    g = e // be
    dfa, dfb = fa.shape[1], fb.shape[1]
    full = lambda i: (0, 0)
    espec = pl.BlockSpec((be, H), lambda i: (i, 0))
    eshape = jax.ShapeDtypeStruct((e, H), jnp.float32)

    def body(fra, waa, wab, frb, wba, wbb, era, erb):
        _ew_body_one(fra, waa, wab, era)
        _ew_body_one(frb, wba, wbb, erb)

    return pl.pallas_call(
        body,
        grid=(g,),
        in_specs=[
            pl.BlockSpec((be, dfa), lambda i: (i, 0)),
            pl.BlockSpec((dfa, MID), full),
            pl.BlockSpec((MID, H), full),
            pl.BlockSpec((be, dfb), lambda i: (i, 0)),
            pl.BlockSpec((dfb, MID), full),
            pl.BlockSpec((MID, H), full),
        ],
        out_specs=[espec, espec],
        out_shape=[eshape, eshape],
    )(fa, wa1, wa2, fb, wb1, wb2)


@functools.lru_cache(maxsize=None)
def _make_sc(n, e, nk=3):
    nc = 2
    nt = 16
    epc = e // nc
    ept = epc // nt
    ch = 48
    nfull = ept // ch
    tl = ept - nfull * ch
    ntrip = (nfull - 1) // 3
    assert nfull - 1 == 3 * ntrip and ntrip >= 2 and tl % 8 == 0
    n_pad = ((n + nt * 8 - 1) // (nt * 8)) * nt * 8
    npt = n_pad // nt
    mesh = plsc.VectorSubcoreMesh(core_axis_name="c", subcore_axis_name="s")
    out_t = jax.ShapeDtypeStruct((nc, n_pad, H), jnp.float32)

    @functools.partial(
        pl.kernel,
        mesh=mesh,
        out_type=[out_t] * nk,
        scratch_types=[
            pltpu.VMEM((ept,), jnp.int32),
            pltpu.VMEM((ch,), jnp.int32),
            pltpu.VMEM((ch,), jnp.int32),
            pltpu.VMEM((ch,), jnp.int32),
            pltpu.VMEM((tl,), jnp.int32),
            pltpu.VMEM((ch, H), jnp.float32),
            pltpu.VMEM((ch, H), jnp.float32),
            pltpu.VMEM((ch, H), jnp.float32),
            pltpu.VMEM((ch, H), jnp.float32),
            pltpu.VMEM((ch, H), jnp.float32),
            pltpu.VMEM((ch, H), jnp.float32),
            pltpu.VMEM_SHARED((n_pad, H), jnp.float32),
        ] + [pltpu.SemaphoreType.DMA] * 12,
    )
    def sc_conv(*refs):
        src_hbm, dst_hbm, zeros_hbm, x1_hbm = refs[:4]
        ew_hbms = refs[4:4 + nk]
        outs = refs[4 + nk:4 + 2 * nk]
        rest = refs[4 + 2 * nk:]
        srcall = rest[0]
        dstb = rest[1:4]
        dstT = rest[4]
        xr = rest[5:8]
        ewb = rest[8:11]
        acc = rest[11]
        gs = rest[12:15]
        es = rest[15:18]
        ds_ = rest[18:21]
        ss = rest[21:24]
        c = lax.axis_index("c")
        s = lax.axis_index("s")
        rows = pl.ds(pl.multiple_of(s * npt, 8), npt)
        tbase = pl.multiple_of(c * epc + s * ept, 8)

        pltpu.sync_copy(src_hbm.at[pl.ds(tbase, ept)], srcall)

        def issue_loads(ew_hbm, ci, q):
            off = pl.multiple_of(ci * ch, 8)
            pltpu.make_async_copy(
                x1_hbm.at[srcall.at[pl.ds(off, ch)]], xr[q], gs[q]).start()
            ebase = pl.multiple_of(tbase + ci * ch, 8)
            pltpu.make_async_copy(
                ew_hbm.at[pl.ds(ebase, ch)], ewb[q], es[q]).start()
            pltpu.make_async_copy(
                dst_hbm.at[pl.ds(ebase, ch)], dstb[q], ds_[q]).start()

        def multiply(xrb, eb, m):
            def two_rows(r2, cc):
                for u in range(2):
                    for j in range(H // 16):
                        sl = pl.ds(j * 16, 16)
                        eb[2 * r2 + u, sl] = eb[2 * r2 + u, sl] * xrb[2 * r2 + u, sl]
                return cc
            lax.fori_loop(0, m // 2, two_rows, 0)

        def scatter_start(q):
            pltpu.async_copy(ewb[q], acc.at[dstb[q]], ss[q], add=True)

        def scatter_wait(q):
            pltpu.make_async_copy(ewb[q], acc.at[dstb[q]], ss[q]).wait()

        def process(ew_hbm, ci, q, qn, issue_ci, do_wait, do_issue):
            pltpu.make_async_copy(
                x1_hbm.at[srcall.at[pl.ds(0, ch)]], xr[q], gs[q]).wait()
            pltpu.make_async_copy(
                ew_hbm.at[pl.ds(0, ch)], ewb[q], es[q]).wait()
            multiply(xr[q], ewb[q], ch)
            pltpu.make_async_copy(
                dst_hbm.at[pl.ds(0, ch)], dstb[q], ds_[q]).wait()
            scatter_start(q)
            if do_wait:
                scatter_wait(qn)
            if do_issue:
                issue_loads(ew_hbm, issue_ci, qn)

        for ew_hbm, out in zip(ew_hbms, outs):
            issue_loads(ew_hbm, 0, 0)
            issue_loads(ew_hbm, 1, 1)
            pltpu.sync_copy(zeros_hbm, acc.at[rows])
            plsc.subcore_barrier()

            process(ew_hbm, 0, 0, 2, 2, False, True)

            def triple(i, carry, ew_hbm=ew_hbm):
                j = 3 * i + 1
                process(ew_hbm, j, 1, 0, j + 2, True, True)
                process(ew_hbm, j + 1, 2, 1, j + 3, True, True)
                process(ew_hbm, j + 2, 0, 2, j + 4, True, True)
                return carry

            lax.fori_loop(0, ntrip - 1, triple, 0)
            jl = nfull - 3
            process(ew_hbm, jl, 1, 0, jl + 2, True, True)
            process(ew_hbm, jl + 1, 2, 1, 0, True, False)
            process(ew_hbm, jl + 2, 0, 2, 0, True, False)
            scatter_wait(0)

            toff = nfull * ch
            tb = pl.multiple_of(tbase + toff, 8)
            pltpu.sync_copy(dst_hbm.at[pl.ds(tb, tl)], dstT)
            pltpu.async_copy(
                x1_hbm.at[srcall.at[pl.ds(toff, tl)]],
                xr[0].at[pl.ds(0, tl)], gs[0]).wait()
            pltpu.sync_copy(ew_hbm.at[pl.ds(tb, tl)], ewb[0].at[pl.ds(0, tl)])
            multiply(xr[0], ewb[0], tl)
            pltpu.sync_copy(ewb[0].at[pl.ds(0, tl)], acc.at[dstT], add=True)

            plsc.subcore_barrier()
            pltpu.sync_copy(acc.at[rows], out.at[c, rows])

    return sc_conv


def _tail_body(a0_ref, a1_ref, a2_ref, x1_ref, x2_ref,
               wrel0, brel0, wroot0, wo0, bo0,
               wrel1, brel1, wroot1, wo1, bo1,
               wrel2, brel2, wroot2, wo2, bo2,
               wc0, bc0, wc1, bc1, wc2, bc2,
               ws0, bs0, ws1, bs1, wfin, bfin,
               out_ref):
    x1 = x1_ref[...]
    x2 = x2_ref[...]
    hs = []
    for ar, wrel, brel, wroot, wo, bo in (
        (a0_ref, wrel0, brel0, wroot0, wo0, bo0),
        (a1_ref, wrel1, brel1, wroot1, wo1, bo1),
        (a2_ref, wrel2, brel2, wroot2, wo2, bo2),
    ):
        agg = ar[0] + ar[1]
        t = (jnp.dot(agg, wrel[...], preferred_element_type=jnp.float32)
             + jnp.dot(x1, wroot[...], preferred_element_type=jnp.float32)
             + brel[...])
        hs.append(_swish(jnp.dot(t, wo[...], preferred_element_type=jnp.float32)
                         + bo[...]))
    wc0v = wc0[...]
    u = (jnp.dot(hs[0], wc0v[:H, :], preferred_element_type=jnp.float32)
         + jnp.dot(hs[1], wc0v[H:2 * H, :], preferred_element_type=jnp.float32)
         + jnp.dot(hs[2], wc0v[2 * H:, :], preferred_element_type=jnp.float32)
         + bc0[...])
    h = _swish(u)
    h = _swish(jnp.dot(h, wc1[...], preferred_element_type=jnp.float32) + bc1[...])
    h = _swish(jnp.dot(h, wc2[...], preferred_element_type=jnp.float32) + bc2[...])
    h = h + x2
    h = _swish(jnp.dot(h, ws0[...], preferred_element_type=jnp.float32) + bs0[...])
    h = _swish(jnp.dot(h, ws1[...], preferred_element_type=jnp.float32) + bs1[...])
    out_ref[...] = jnp.dot(h, wfin[...], preferred_element_type=jnp.float32) + bfin[...]


def _tail(a0, a1, a2, x1, x2, p):
    n = x1.shape[0]
    bn = 1000
    g = n // bn
    full = lambda i: (0, 0)
    aspec = pl.BlockSpec((2, bn, H), lambda i: (0, i, 0))
    nspec = pl.BlockSpec((bn, H), lambda i: (i, 0))
    wspec = pl.BlockSpec((H, H), full)
    bspec = pl.BlockSpec((1, H), full)
    b2 = lambda v: v.reshape(1, H)
    args = [a0, a1, a2, x1, x2]
    specs = [aspec, aspec, aspec, nspec, nspec]
    for k in ("c0", "c1", "c2"):
        args += [p[k + "_wrel"], b2(p[k + "_brel"]), p[k + "_wroot"]]
        specs += [wspec, bspec, wspec]
        o = "o" + k[1]
        args += [p["w_" + o], b2(p["b_" + o])]
        specs += [wspec, bspec]
    args += [p["wc0"], b2(p["bc0"]), p["wc1"], b2(p["bc1"]),
             p["wc2"], b2(p["bc2"]), p["ws0"], b2(p["bs0"]),
             p["ws1"], b2(p["bs1"]), p["w_fin"], b2(p["b_fin"])]
    specs += [pl.BlockSpec((3 * H, H), full), bspec, wspec, bspec,
              wspec, bspec, wspec, bspec, wspec, bspec, wspec, bspec]
    return pl.pallas_call(
        _tail_body,
        grid=(g,),
        in_specs=specs,
        out_specs=pl.BlockSpec((bn, H), lambda i: (i, 0)),
        out_shape=jax.ShapeDtypeStruct((n, H), jnp.float32),
    )(*args)


def kernel(x, feature0, feature1, pos_emb, edge_index, batch, params):
    del batch
    p = params
    n, e = x.shape[0], feature0.shape[0]
    x1, x2 = _pre(x, p["w_h1"], p["b_h1"].reshape(1, H),
                  p["w_h2"], p["b_h2"].reshape(1, H))
    n_pad = ((n + 127) // 128) * 128
    zeros = jnp.zeros((n_pad // 16, H), jnp.float32)
    sc1 = _make_sc(n, e, 1)
    sc2 = _make_sc(n, e, 2)
    src, dst = edge_index[0], edge_index[1]
    ew0 = _ew_one(feature0, p["f0_w1"], p["f0_w2"])
    (a0,) = sc1(src, dst, zeros, x1, ew0)
    ew1, ew2 = _ew_two(feature1, p["f1_w1"], p["f1_w2"],
                       pos_emb, p["f2_w1"], p["f2_w2"])
    a1, a2 = sc2(src, dst, zeros, x1, ew1, ew2)
    return _tail(a0, a1, a2, x1, x2, p)


# speedup vs baseline: 1.0355x; 1.0259x over previous
import functools

import jax
import jax.numpy as jnp
from jax import lax
from jax.experimental import pallas as pl
from jax.experimental.pallas import tpu as pltpu
from jax.experimental.pallas import tpu_sc as plsc

H = 128
MID = 64


def _swish(v):
    return v * jax.nn.sigmoid(v)


def _pre_body(x_ref, wh1_ref, bh1_ref, wh2_ref, bh2_ref, x1_ref, x2_ref):
    xb = x_ref[...]
    x1_ref[...] = _swish(
        jnp.dot(xb, wh1_ref[...], preferred_element_type=jnp.float32)
        + bh1_ref[...])
    x2_ref[...] = _swish(
        jnp.dot(xb, wh2_ref[...], preferred_element_type=jnp.float32)
        + bh2_ref[...])


def _pre(x, wh1, bh1, wh2, bh2):
    n = x.shape[0]
    bn = 1000
    g = n // bn
    full = lambda i: (0, 0)
    nspec = pl.BlockSpec((bn, H), lambda i: (i, 0))
    return pl.pallas_call(
        _pre_body,
        grid=(g,),
        in_specs=[
            nspec,
            pl.BlockSpec((H, H), full),
            pl.BlockSpec((1, H), full),
            pl.BlockSpec((H, H), full),
            pl.BlockSpec((1, H), full),
        ],
        out_specs=[nspec, nspec],
        out_shape=[
            jax.ShapeDtypeStruct((n, H), jnp.float32),
            jax.ShapeDtypeStruct((n, H), jnp.float32),
        ],
    )(x, wh1, bh1, wh2, bh2)


def _ew_one(feat, w1, w2):
    e = feat.shape[0]
    be = 4000
    g = e // be
    f = feat.shape[1]
    full = lambda i: (0, 0)
    return pl.pallas_call(
        lambda fr, wa, wb, er: _ew_body_one(fr, wa, wb, er),
        grid=(g,),
        in_specs=[
            pl.BlockSpec((be, f), lambda i: (i, 0)),
            pl.BlockSpec((f, MID), full),
            pl.BlockSpec((MID, H), full),
        ],
        out_specs=pl.BlockSpec((be, H), lambda i: (i, 0)),
        out_shape=jax.ShapeDtypeStruct((e, H), jnp.float32),
    )(feat, w1, w2)


def _ew_body_one(fr, wa, wb, er):
    mid = jnp.dot(fr[...], wa[...], preferred_element_type=jnp.float32)
    er[...] = jnp.dot(mid, wb[...], preferred_element_type=jnp.float32)


def _ew_two(fa, wa1, wa2, fb, wb1, wb2):
    e = fa.shape[0]
    be = 4000
    g = e // be
    dfa, dfb = fa.shape[1], fb.shape[1]
    full = lambda i: (0, 0)
    espec = pl.BlockSpec((be, H), lambda i: (i, 0))
    eshape = jax.ShapeDtypeStruct((e, H), jnp.float32)

    def body(fra, waa, wab, frb, wba, wbb, era, erb):
        _ew_body_one(fra, waa, wab, era)
        _ew_body_one(frb, wba, wbb, erb)

    return pl.pallas_call(
        body,
        grid=(g,),
        in_specs=[
            pl.BlockSpec((be, dfa), lambda i: (i, 0)),
            pl.BlockSpec((dfa, MID), full),
            pl.BlockSpec((MID, H), full),
            pl.BlockSpec((be, dfb), lambda i: (i, 0)),
            pl.BlockSpec((dfb, MID), full),
            pl.BlockSpec((MID, H), full),
        ],
        out_specs=[espec, espec],
        out_shape=[eshape, eshape],
    )(fa, wa1, wa2, fb, wb1, wb2)


@functools.lru_cache(maxsize=None)
def _make_sc(n, e, nk=3):
    nc = 2
    nt = 16
    epc = e // nc
    ept = epc // nt
    ch = 48
    nfull = ept // ch
    tl = ept - nfull * ch
    ntrip = (nfull - 1) // 3
    assert nfull - 1 == 3 * ntrip and ntrip >= 2 and tl % 8 == 0
    n_pad = ((n + nt * 8 - 1) // (nt * 8)) * nt * 8
    npt = n_pad // nt
    mesh = plsc.VectorSubcoreMesh(core_axis_name="c", subcore_axis_name="s")
    out_t = jax.ShapeDtypeStruct((nc, n_pad, H), jnp.float32)

    @functools.partial(
        pl.kernel,
        mesh=mesh,
        out_type=[out_t] * nk,
        scratch_types=[
            pltpu.VMEM((ept,), jnp.int32),
            pltpu.VMEM((ch,), jnp.int32),
            pltpu.VMEM((ch,), jnp.int32),
            pltpu.VMEM((ch,), jnp.int32),
            pltpu.VMEM((tl,), jnp.int32),
            pltpu.VMEM((ch, H), jnp.float32),
            pltpu.VMEM((ch, H), jnp.float32),
            pltpu.VMEM((ch, H), jnp.float32),
            pltpu.VMEM((ch, H), jnp.float32),
            pltpu.VMEM((ch, H), jnp.float32),
            pltpu.VMEM((ch, H), jnp.float32),
            pltpu.VMEM_SHARED((n_pad, H), jnp.float32),
        ] + [pltpu.SemaphoreType.DMA] * 12,
    )
    def sc_conv(*refs):
        src_hbm, dst_hbm, zeros_hbm, x1_hbm = refs[:4]
        ew_hbms = refs[4:4 + nk]
        outs = refs[4 + nk:4 + 2 * nk]
        rest = refs[4 + 2 * nk:]
        srcall = rest[0]
        dstb = rest[1:4]
        dstT = rest[4]
        xr = rest[5:8]
        ewb = rest[8:11]
        acc = rest[11]
        gs = rest[12:15]
        es = rest[15:18]
        ds_ = rest[18:21]
        ss = rest[21:24]
        c = lax.axis_index("c")
        s = lax.axis_index("s")
        rows = pl.ds(pl.multiple_of(s * npt, 8), npt)
        tbase = pl.multiple_of(c * epc + s * ept, 8)

        pltpu.sync_copy(src_hbm.at[pl.ds(tbase, ept)], srcall)

        def issue_loads(ew_hbm, ci, q):
            off = pl.multiple_of(ci * ch, 8)
            pltpu.make_async_copy(
                x1_hbm.at[srcall.at[pl.ds(off, ch)]], xr[q], gs[q]).start()
            ebase = pl.multiple_of(tbase + ci * ch, 8)
            pltpu.make_async_copy(
                ew_hbm.at[pl.ds(ebase, ch)], ewb[q], es[q]).start()
            pltpu.make_async_copy(
                dst_hbm.at[pl.ds(ebase, ch)], dstb[q], ds_[q]).start()

        def multiply(xrb, eb, m):
            def two_rows(r2, cc):
                for u in range(2):
                    for j in range(H // 16):
                        sl = pl.ds(j * 16, 16)
                        eb[2 * r2 + u, sl] = eb[2 * r2 + u, sl] * xrb[2 * r2 + u, sl]
                return cc
            lax.fori_loop(0, m // 2, two_rows, 0)

        def scatter_start(q):
            pltpu.async_copy(ewb[q], acc.at[dstb[q]], ss[q], add=True)

        def scatter_wait(q):
            pltpu.make_async_copy(ewb[q], acc.at[dstb[q]], ss[q]).wait()

        def process(ew_hbm, ci, q, qn, issue_ci, do_wait, do_issue):
            pltpu.make_async_copy(
                x1_hbm.at[srcall.at[pl.ds(0, ch)]], xr[q], gs[q]).wait()
            pltpu.make_async_copy(
                ew_hbm.at[pl.ds(0, ch)], ewb[q], es[q]).wait()
            multiply(xr[q], ewb[q], ch)
            pltpu.make_async_copy(
                dst_hbm.at[pl.ds(0, ch)], dstb[q], ds_[q]).wait()
            scatter_start(q)
            if do_wait:
                scatter_wait(qn)
            if do_issue:
                issue_loads(ew_hbm, issue_ci, qn)

        for ew_hbm, out in zip(ew_hbms, outs):
            issue_loads(ew_hbm, 0, 0)
            issue_loads(ew_hbm, 1, 1)
            pltpu.sync_copy(zeros_hbm, acc.at[rows])
            plsc.subcore_barrier()

            process(ew_hbm, 0, 0, 2, 2, False, True)

            def triple(i, carry, ew_hbm=ew_hbm):
                j = 3 * i + 1
                process(ew_hbm, j, 1, 0, j + 2, True, True)
                process(ew_hbm, j + 1, 2, 1, j + 3, True, True)
                process(ew_hbm, j + 2, 0, 2, j + 4, True, True)
                return carry

            lax.fori_loop(0, ntrip - 1, triple, 0)
            jl = nfull - 3
            process(ew_hbm, jl, 1, 0, jl + 2, True, True)
            process(ew_hbm, jl + 1, 2, 1, 0, True, False)
            process(ew_hbm, jl + 2, 0, 2, 0, True, False)
            scatter_wait(0)

            toff = nfull * ch
            tb = pl.multiple_of(tbase + toff, 8)
            pltpu.sync_copy(dst_hbm.at[pl.ds(tb, tl)], dstT)
            pltpu.async_copy(
                x1_hbm.at[srcall.at[pl.ds(toff, tl)]],
                xr[0].at[pl.ds(0, tl)], gs[0]).wait()
            pltpu.sync_copy(ew_hbm.at[pl.ds(tb, tl)], ewb[0].at[pl.ds(0, tl)])
            multiply(xr[0], ewb[0], tl)
            pltpu.sync_copy(ewb[0].at[pl.ds(0, tl)], acc.at[dstT], add=True)

            plsc.subcore_barrier()
            pltpu.sync_copy(acc.at[rows], out.at[c, rows])

    return sc_conv


def _tail_body(a0_ref, a1_ref, a2_ref, x1_ref, x2_ref,
               wrel0, brel0, wroot0, wo0, bo0,
               wrel1, brel1, wroot1, wo1, bo1,
               wrel2, brel2, wroot2, wo2, bo2,
               wc0, bc0, wc1, bc1, wc2, bc2,
               ws0, bs0, ws1, bs1, wfin, bfin,
               out_ref):
    x1 = x1_ref[...]
    x2 = x2_ref[...]
    hs = []
    for ar, wrel, brel, wroot, wo, bo in (
        (a0_ref, wrel0, brel0, wroot0, wo0, bo0),
        (a1_ref, wrel1, brel1, wroot1, wo1, bo1),
        (a2_ref, wrel2, brel2, wroot2, wo2, bo2),
    ):
        agg = ar[0] + ar[1]
        t = (jnp.dot(agg, wrel[...], preferred_element_type=jnp.float32)
             + jnp.dot(x1, wroot[...], preferred_element_type=jnp.float32)
             + brel[...])
        hs.append(_swish(jnp.dot(t, wo[...], preferred_element_type=jnp.float32)
                         + bo[...]))
    wc0v = wc0[...]
    u = (jnp.dot(hs[0], wc0v[:H, :], preferred_element_type=jnp.float32)
         + jnp.dot(hs[1], wc0v[H:2 * H, :], preferred_element_type=jnp.float32)
         + jnp.dot(hs[2], wc0v[2 * H:, :], preferred_element_type=jnp.float32)
         + bc0[...])
    h = _swish(u)
    h = _swish(jnp.dot(h, wc1[...], preferred_element_type=jnp.float32) + bc1[...])
    h = _swish(jnp.dot(h, wc2[...], preferred_element_type=jnp.float32) + bc2[...])
    h = h + x2
    h = _swish(jnp.dot(h, ws0[...], preferred_element_type=jnp.float32) + bs0[...])
    h = _swish(jnp.dot(h, ws1[...], preferred_element_type=jnp.float32) + bs1[...])
    out_ref[...] = jnp.dot(h, wfin[...], preferred_element_type=jnp.float32) + bfin[...]


def _tail(a0, a1, a2, x1, x2, p):
    n = x1.shape[0]
    bn = 1000
    g = n // bn
    full = lambda i: (0, 0)
    aspec = pl.BlockSpec((2, bn, H), lambda i: (0, i, 0))
    nspec = pl.BlockSpec((bn, H), lambda i: (i, 0))
    wspec = pl.BlockSpec((H, H), full)
    bspec = pl.BlockSpec((1, H), full)
    b2 = lambda v: v.reshape(1, H)
    args = [a0, a1, a2, x1, x2]
    specs = [aspec, aspec, aspec, nspec, nspec]
    for k in ("c0", "c1", "c2"):
        args += [p[k + "_wrel"], b2(p[k + "_brel"]), p[k + "_wroot"]]
        specs += [wspec, bspec, wspec]
        o = "o" + k[1]
        args += [p["w_" + o], b2(p["b_" + o])]
        specs += [wspec, bspec]
    args += [p["wc0"], b2(p["bc0"]), p["wc1"], b2(p["bc1"]),
             p["wc2"], b2(p["bc2"]), p["ws0"], b2(p["bs0"]),
             p["ws1"], b2(p["bs1"]), p["w_fin"], b2(p["b_fin"])]
    specs += [pl.BlockSpec((3 * H, H), full), bspec, wspec, bspec,
              wspec, bspec, wspec, bspec, wspec, bspec, wspec, bspec]
    return pl.pallas_call(
        _tail_body,
        grid=(g,),
        in_specs=specs,
        out_specs=pl.BlockSpec((bn, H), lambda i: (i, 0)),
        out_shape=jax.ShapeDtypeStruct((n, H), jnp.float32),
    )(*args)


def kernel(x, feature0, feature1, pos_emb, edge_index, batch, params):
    del batch
    p = params
    n, e = x.shape[0], feature0.shape[0]
    x1, x2 = _pre(x, p["w_h1"], p["b_h1"].reshape(1, H),
                  p["w_h2"], p["b_h2"].reshape(1, H))
    n_pad = ((n + 127) // 128) * 128
    zeros = jnp.zeros((n_pad // 16, H), jnp.float32)
    sc1 = _make_sc(n, e, 1)
    sc2 = _make_sc(n, e, 2)
    src, dst = edge_index[0], edge_index[1]
    ew0 = _ew_one(feature0, p["f0_w1"], p["f0_w2"])
    (a0,) = sc1(src, dst, zeros, x1, ew0)
    ew1, ew2 = _ew_two(feature1, p["f1_w1"], p["f1_w2"],
                       pos_emb, p["f2_w1"], p["f2_w2"])
    a1, a2 = sc2(src, dst, zeros, x1, ew1, ew2)
    return _tail(a0, a1, a2, x1, x2, p)
